# baseline (speedup 1.0000x reference)
import jax
import jax.numpy as jnp
from jax.experimental import pallas as pl

N = 100000
D = 32


def _bias_relu_kernel(x_ref, b_ref, o_ref):
    o_ref[...] = jnp.maximum(x_ref[...] + b_ref[...], 0.0)


def _bias_relu(x, b):
    RB = 1000
    return pl.pallas_call(
        _bias_relu_kernel,
        grid=(N // RB,),
        in_specs=[
            pl.BlockSpec((RB, D), lambda i: (i, 0)),
            pl.BlockSpec((1, D), lambda i: (0, 0)),
        ],
        out_specs=pl.BlockSpec((RB, D), lambda i: (i, 0)),
        out_shape=jax.ShapeDtypeStruct((N, D), jnp.float32),
    )(x, b.reshape(1, D))


def kernel(adj_indices, adj_values, emb_table, W0, b0, W1, b1):
    dst = adj_indices[0]
    src = adj_indices[1]
    emb = emb_table
    for W, b in ((W0, b0), (W1, b1)):
        msg = adj_values[:, None] * jnp.take(emb, src, axis=0)
        agg = jax.ops.segment_sum(msg, dst, num_segments=N)
        emb = _bias_relu(agg @ W.T, b)
    return emb


# speedup vs baseline: 7.2649x; 7.2649x over previous
import functools

import jax
import jax.numpy as jnp
from jax import lax
from jax.experimental import pallas as pl
from jax.experimental.pallas import tpu as pltpu
from jax.experimental.pallas import tpu_sc as plsc

N = 100000
D = 32
HALF = 16
NC = 2
NS = 16

BLK = 128
INNER = 8
SUPER = INNER * BLK

ROWS_PER_SUB = 6256
ZROWS = 128


_SPLAT_DNUMS = lax.GatherDimensionNumbers(
    offset_dims=(), collapsed_slice_dims=(0,), start_index_map=(0,))


def _splat(v16, e):
    idx = jnp.full((16, 1), e, jnp.int32)
    return lax.gather(v16, idx, _SPLAT_DNUMS, (1,),
                      mode=lax.GatherScatterMode.PROMISE_IN_BOUNDS)


def _agg_body(emb_hbm, src_hbm, dst_hbm, val_hbm, out_hbm,
              agg_sh, src_v, dst_v, val_v, rows_v, zbuf, gsem,
              *, n_super):
    c = lax.axis_index("c")
    s = lax.axis_index("s")

    def zero_zbuf(i, carry):
        zbuf[i, :] = jnp.zeros((HALF,), jnp.float32)
        return carry

    lax.fori_loop(0, ZROWS, zero_zbuf, 0)
    r0 = s * ROWS_PER_SUB - (s // (NS - 1)) * (NS * ROWS_PER_SUB - N)
    for t in range(ROWS_PER_SUB // ZROWS):
        pltpu.sync_copy(zbuf, agg_sh.at[pl.ds(r0 + t * ZROWS, ZROWS)])
    rem = ROWS_PER_SUB % ZROWS
    if rem:
        pltpu.sync_copy(zbuf.at[pl.ds(0, rem)],
                        agg_sh.at[pl.ds(r0 + ROWS_PER_SUB - rem, rem)])
    plsc.subcore_barrier()

    row_base = s * (n_super * INNER)

    def super_body(g, carry):
        rb = row_base + g * INNER
        pltpu.sync_copy(src_hbm.at[pl.ds(rb, INNER)], src_v)
        pltpu.sync_copy(dst_hbm.at[pl.ds(rb, INNER)], dst_v)
        pltpu.sync_copy(val_hbm.at[pl.ds(rb, INNER)], val_v)

        for j in range(INNER):
            def xform(k, carry2):
                sl = pl.ds(k * 16, 16)
                src_v[j, sl] = src_v[j, sl] * 2 + c
                return carry2

            lax.fori_loop(0, BLK // 16, xform, 0)

            pltpu.async_copy(emb_hbm.at[src_v.at[j]], rows_v.at[j], gsem).wait()

            def scale(k, carry2):
                v16 = val_v[j, pl.ds(k * 16, 16)]
                for e in range(16):
                    r = k * 16 + e
                    rows_v[j, r, :] = rows_v[j, r, :] * _splat(v16, e)
                return carry2

            lax.fori_loop(0, BLK // 16, scale, 0)

            pltpu.sync_copy(rows_v.at[j], agg_sh.at[dst_v.at[j]], add=True)
        return carry

    lax.fori_loop(0, n_super, super_body, 0)
    plsc.subcore_barrier()

    pltpu.sync_copy(agg_sh.at[pl.ds(r0, ROWS_PER_SUB)],
                    out_hbm.at[pl.ds(c * N + r0, ROWS_PER_SUB)])


@functools.partial(jax.jit, static_argnames=("n_super",))
def _sc_aggregate(emb_r, src2, dst2, val2, n_super):
    mesh = plsc.VectorSubcoreMesh(core_axis_name="c", subcore_axis_name="s",
                                  num_cores=NC, num_subcores=NS)
    return pl.kernel(
        functools.partial(_agg_body, n_super=n_super),
        out_type=jax.ShapeDtypeStruct((2 * N, HALF), jnp.float32),
        mesh=mesh,
        scratch_types=[
            pltpu.VMEM_SHARED((N, HALF), jnp.float32),
            pltpu.VMEM((INNER, BLK), jnp.int32),
            pltpu.VMEM((INNER, BLK), jnp.int32),
            pltpu.VMEM((INNER, BLK), jnp.float32),
            pltpu.VMEM((INNER, BLK, HALF), jnp.float32),
            pltpu.VMEM((ZROWS, HALF), jnp.float32),
            pltpu.SemaphoreType.DMA,
        ],
        compiler_params=pltpu.CompilerParams(use_tc_tiling_on_sc=False),
    )(emb_r, src2, dst2, val2)


def _linear_kernel(lo_ref, hi_ref, w_ref, b_ref, o_ref):
    x = jnp.concatenate([lo_ref[...], hi_ref[...]], axis=1)
    h = lax.dot_general(x, w_ref[...], (((1,), (1,)), ((), ())),
                        preferred_element_type=jnp.float32)
    o_ref[...] = jnp.maximum(h + b_ref[...], 0.0)


def _linear(agg_r, W, b):
    RB = 1000
    nb = N // RB
    return pl.pallas_call(
        _linear_kernel,
        grid=(nb,),
        in_specs=[
            pl.BlockSpec((RB, HALF), lambda i: (i, 0)),
            pl.BlockSpec((RB, HALF), lambda i: (nb + i, 0)),
            pl.BlockSpec((D, D), lambda i: (0, 0)),
            pl.BlockSpec((1, D), lambda i: (0, 0)),
        ],
        out_specs=pl.BlockSpec((RB, D), lambda i: (i, 0)),
        out_shape=jax.ShapeDtypeStruct((N, D), jnp.float32),
    )(agg_r, agg_r, W, b.reshape(1, D))


def kernel(adj_indices, adj_values, emb_table, W0, b0, W1, b1):
    E = adj_values.shape[0]
    e_unit = NS * SUPER
    E_pad = -(-E // e_unit) * e_unit
    pad = E_pad - E
    if pad:
        pad_idx = (jnp.arange(pad, dtype=jnp.int32) * 97) % N
        src = jnp.concatenate([adj_indices[1], pad_idx])
        dst = jnp.concatenate([adj_indices[0], pad_idx])
        val = jnp.concatenate([adj_values, jnp.zeros((pad,), jnp.float32)])
    else:
        src, dst, val = adj_indices[1], adj_indices[0], adj_values
    src2 = src.reshape(-1, BLK)
    dst2 = dst.reshape(-1, BLK)
    val2 = val.reshape(-1, BLK)
    n_super = E_pad // (NS * SUPER)

    emb_r = emb_table.reshape(2 * N, HALF)
    agg1 = _sc_aggregate(emb_r, src2, dst2, val2, n_super)
    emb1 = _linear(agg1, W0, b0)
    agg2 = _sc_aggregate(emb1.reshape(2 * N, HALF), src2, dst2, val2, n_super)
    return _linear(agg2, W1, b1)


# speedup vs baseline: 15.8227x; 2.1780x over previous
import functools

import jax
import jax.numpy as jnp
from jax import lax
from jax.experimental import pallas as pl
from jax.experimental.pallas import tpu as pltpu
from jax.experimental.pallas import tpu_sc as plsc

N = 100000
D = 32
HALF = 16
NC = 2
NS = 16

BLK = 128
INNER = 8
SUPER = INNER * BLK

ROWS_PER_SUB = 6256
ZROWS = 128


_SPLAT_DNUMS = lax.GatherDimensionNumbers(
    offset_dims=(), collapsed_slice_dims=(0,), start_index_map=(0,))


def _splat(v16, e):
    idx = jnp.full((16, 1), e, jnp.int32)
    return lax.gather(v16, idx, _SPLAT_DNUMS, (1,),
                      mode=lax.GatherScatterMode.PROMISE_IN_BOUNDS)


def _agg_body(emb_hbm, src_hbm, dst_hbm, val_hbm, out_hbm,
              agg_sh, src_v, dst_v, val_v, rows_v, zbuf, gsem, ssem, isem,
              *, n_super):
    c = lax.axis_index("c")
    s = lax.axis_index("s")

    def zero_zbuf(i, carry):
        zbuf[i, :] = jnp.zeros((HALF,), jnp.float32)
        return carry

    lax.fori_loop(0, ZROWS, zero_zbuf, 0)
    r0 = s * ROWS_PER_SUB - (s // (NS - 1)) * (NS * ROWS_PER_SUB - N)
    for t in range(ROWS_PER_SUB // ZROWS):
        pltpu.sync_copy(zbuf, agg_sh.at[pl.ds(r0 + t * ZROWS, ZROWS)])
    rem = ROWS_PER_SUB % ZROWS
    if rem:
        pltpu.sync_copy(zbuf.at[pl.ds(0, rem)],
                        agg_sh.at[pl.ds(r0 + ROWS_PER_SUB - rem, rem)])
    plsc.subcore_barrier()

    row_base = s * (n_super * INNER)

    def fire_idx_loads(g, q):
        rb = row_base + g * INNER
        pltpu.async_copy(src_hbm.at[pl.ds(rb, INNER)], src_v.at[q], isem)
        pltpu.async_copy(dst_hbm.at[pl.ds(rb, INNER)], dst_v.at[q], isem)
        pltpu.async_copy(val_hbm.at[pl.ds(rb, INNER)], val_v.at[q], isem)

    def wait_idx_loads(g, q):
        rb = row_base + g * INNER
        pltpu.make_async_copy(src_hbm.at[pl.ds(rb, INNER)], src_v.at[q], isem).wait()
        pltpu.make_async_copy(dst_hbm.at[pl.ds(rb, INNER)], dst_v.at[q], isem).wait()
        pltpu.make_async_copy(val_hbm.at[pl.ds(rb, INNER)], val_v.at[q], isem).wait()

    def xform_and_fire_gathers(q):
        for j in range(INNER):
            for k in range(BLK // 16):
                sl = pl.ds(k * 16, 16)
                src_v[q, j, sl] = src_v[q, j, sl] * 2 + c
            pltpu.async_copy(emb_hbm.at[src_v.at[q, j]], rows_v.at[j], gsem)

    def stage(g, p, prefetch_pred):
        q = 1 - p
        for j in range(INNER):
            pltpu.make_async_copy(emb_hbm.at[src_v.at[p, j]], rows_v.at[j],
                                  gsem).wait()

            def scale(k, carry2):
                v16 = val_v[p, j, pl.ds(k * 16, 16)]
                for e in range(16):
                    r = k * 16 + e
                    rows_v[j, r, :] = rows_v[j, r, :] * _splat(v16, e)
                return carry2

            lax.fori_loop(0, BLK // 16, scale, 0)

            pltpu.async_copy(rows_v.at[j], agg_sh.at[dst_v.at[p, j]], ssem,
                             add=True)

        def next_super():
            fire_idx_loads(g + 1, q)

        if prefetch_pred is None:
            next_super()
        else:
            pl.when(prefetch_pred)(next_super)

        for j in range(INNER):
            pltpu.make_async_copy(rows_v.at[j], agg_sh.at[dst_v.at[p, j]],
                                  ssem).wait()

        def launch():
            wait_idx_loads(g + 1, q)
            xform_and_fire_gathers(q)

        if prefetch_pred is None:
            launch()
        else:
            pl.when(prefetch_pred)(launch)

    pltpu.sync_copy(src_hbm.at[pl.ds(row_base, INNER)], src_v.at[0])
    pltpu.sync_copy(dst_hbm.at[pl.ds(row_base, INNER)], dst_v.at[0])
    pltpu.sync_copy(val_hbm.at[pl.ds(row_base, INNER)], val_v.at[0])
    xform_and_fire_gathers(0)

    def super_body2(g2, carry):
        stage(2 * g2, 0, None)
        stage(2 * g2 + 1, 1, g2 < n_super // 2 - 1)
        return carry

    lax.fori_loop(0, n_super // 2, super_body2, 0)
    plsc.subcore_barrier()

    pltpu.sync_copy(agg_sh.at[pl.ds(r0, ROWS_PER_SUB)],
                    out_hbm.at[pl.ds(c * N + r0, ROWS_PER_SUB)])


@functools.partial(jax.jit, static_argnames=("n_super",))
def _sc_aggregate(emb_r, src2, dst2, val2, n_super):
    mesh = plsc.VectorSubcoreMesh(core_axis_name="c", subcore_axis_name="s",
                                  num_cores=NC, num_subcores=NS)
    return pl.kernel(
        functools.partial(_agg_body, n_super=n_super),
        out_type=jax.ShapeDtypeStruct((2 * N, HALF), jnp.float32),
        mesh=mesh,
        scratch_types=[
            pltpu.VMEM_SHARED((N, HALF), jnp.float32),
            pltpu.VMEM((2, INNER, BLK), jnp.int32),
            pltpu.VMEM((2, INNER, BLK), jnp.int32),
            pltpu.VMEM((2, INNER, BLK), jnp.float32),
            pltpu.VMEM((INNER, BLK, HALF), jnp.float32),
            pltpu.VMEM((ZROWS, HALF), jnp.float32),
            pltpu.SemaphoreType.DMA,
            pltpu.SemaphoreType.DMA,
            pltpu.SemaphoreType.DMA,
        ],
        compiler_params=pltpu.CompilerParams(use_tc_tiling_on_sc=False),
    )(emb_r, src2, dst2, val2)


def _linear_kernel(lo_ref, hi_ref, w_ref, b_ref, o_ref):
    x = jnp.concatenate([lo_ref[...], hi_ref[...]], axis=1)
    h = lax.dot_general(x, w_ref[...], (((1,), (1,)), ((), ())),
                        preferred_element_type=jnp.float32)
    o_ref[...] = jnp.maximum(h + b_ref[...], 0.0)


def _linear(agg_r, W, b):
    RB = 1000
    nb = N // RB
    return pl.pallas_call(
        _linear_kernel,
        grid=(nb,),
        in_specs=[
            pl.BlockSpec((RB, HALF), lambda i: (i, 0)),
            pl.BlockSpec((RB, HALF), lambda i: (nb + i, 0)),
            pl.BlockSpec((D, D), lambda i: (0, 0)),
            pl.BlockSpec((1, D), lambda i: (0, 0)),
        ],
        out_specs=pl.BlockSpec((RB, D), lambda i: (i, 0)),
        out_shape=jax.ShapeDtypeStruct((N, D), jnp.float32),
    )(agg_r, agg_r, W, b.reshape(1, D))


def kernel(adj_indices, adj_values, emb_table, W0, b0, W1, b1):
    E = adj_values.shape[0]
    e_unit = NS * SUPER
    E_pad = -(-E // e_unit) * e_unit
    pad = E_pad - E
    if pad:
        pad_idx = (jnp.arange(pad, dtype=jnp.int32) * 97) % N
        src = jnp.concatenate([adj_indices[1], pad_idx])
        dst = jnp.concatenate([adj_indices[0], pad_idx])
        val = jnp.concatenate([adj_values, jnp.zeros((pad,), jnp.float32)])
    else:
        src, dst, val = adj_indices[1], adj_indices[0], adj_values
    src2 = src.reshape(-1, BLK)
    dst2 = dst.reshape(-1, BLK)
    val2 = val.reshape(-1, BLK)
    n_super = E_pad // (NS * SUPER)

    emb_r = emb_table.reshape(2 * N, HALF)
    agg1 = _sc_aggregate(emb_r, src2, dst2, val2, n_super)
    emb1 = _linear(agg1, W0, b0)
    agg2 = _sc_aggregate(emb1.reshape(2 * N, HALF), src2, dst2, val2, n_super)
    return _linear(agg2, W1, b1)


# speedup vs baseline: 21.0796x; 1.3322x over previous
import functools

import jax
import jax.numpy as jnp
from jax import lax
from jax.experimental import pallas as pl
from jax.experimental.pallas import tpu as pltpu
from jax.experimental.pallas import tpu_sc as plsc

N = 100000
D = 32
HALF = 16
NC = 2
NS = 16

BLK = 128
INNER = 8
SUPER = INNER * BLK

ROWS_PER_SUB = 6256
ZROWS = 128


_SPLAT_DNUMS = lax.GatherDimensionNumbers(
    offset_dims=(), collapsed_slice_dims=(0,), start_index_map=(0,))


def _splat(v16, e):
    idx = jnp.full((16, 1), e, jnp.int32)
    return lax.gather(v16, idx, _SPLAT_DNUMS, (1,),
                      mode=lax.GatherScatterMode.PROMISE_IN_BOUNDS)


def _agg_body(emb_hbm, src_hbm, dst_hbm, val_hbm, out_hbm,
              agg_sh, src_v, dst_v, val_v, rows_v, zbuf, gsem, ssem, isem,
              *, n_super):
    c = lax.axis_index("c")
    s = lax.axis_index("s")

    def zero_zbuf(i, carry):
        zbuf[i, :] = jnp.zeros((HALF,), jnp.float32)
        return carry

    lax.fori_loop(0, ZROWS, zero_zbuf, 0)
    r0 = s * ROWS_PER_SUB - (s // (NS - 1)) * (NS * ROWS_PER_SUB - N)
    for t in range(ROWS_PER_SUB // ZROWS):
        pltpu.sync_copy(zbuf, agg_sh.at[pl.ds(r0 + t * ZROWS, ZROWS)])
    rem = ROWS_PER_SUB % ZROWS
    if rem:
        pltpu.sync_copy(zbuf.at[pl.ds(0, rem)],
                        agg_sh.at[pl.ds(r0 + ROWS_PER_SUB - rem, rem)])
    plsc.subcore_barrier()

    row_base = s * (n_super * INNER)

    def fire_idx_loads(g, q):
        rb = row_base + g * INNER
        pltpu.async_copy(src_hbm.at[pl.ds(rb, INNER)], src_v.at[q], isem)
        pltpu.async_copy(dst_hbm.at[pl.ds(rb, INNER)], dst_v.at[q], isem)
        pltpu.async_copy(val_hbm.at[pl.ds(rb, INNER)], val_v.at[q], isem)

    def wait_idx_loads(g, q):
        rb = row_base + g * INNER
        pltpu.make_async_copy(src_hbm.at[pl.ds(rb, INNER)], src_v.at[q], isem).wait()
        pltpu.make_async_copy(dst_hbm.at[pl.ds(rb, INNER)], dst_v.at[q], isem).wait()
        pltpu.make_async_copy(val_hbm.at[pl.ds(rb, INNER)], val_v.at[q], isem).wait()

    def xform_and_fire_gathers(q):
        for j in range(INNER):
            for k in range(BLK // 16):
                sl = pl.ds(k * 16, 16)
                src_v[q, j, sl] = src_v[q, j, sl] * 2 + c
            pltpu.async_copy(emb_hbm.at[src_v.at[q, j]], rows_v.at[j], gsem)

    def stage(g, p, prefetch_pred):
        q = 1 - p
        for j in range(INNER):
            pltpu.make_async_copy(emb_hbm.at[src_v.at[p, j]], rows_v.at[j],
                                  gsem).wait()

            def scale(k, carry2):
                v16 = val_v[p, j, pl.ds(k * 16, 16)]
                for e in range(16):
                    r = k * 16 + e
                    rows_v[j, r, :] = rows_v[j, r, :] * _splat(v16, e)
                return carry2

            lax.fori_loop(0, BLK // 16, scale, 0)

            pltpu.async_copy(rows_v.at[j], agg_sh.at[dst_v.at[p, j]], ssem,
                             add=True)

        def next_super():
            fire_idx_loads(g + 1, q)

        if prefetch_pred is None:
            next_super()
        else:
            pl.when(prefetch_pred)(next_super)

        for j in range(INNER):
            pltpu.make_async_copy(rows_v.at[j], agg_sh.at[dst_v.at[p, j]],
                                  ssem).wait()

        def launch():
            wait_idx_loads(g + 1, q)
            xform_and_fire_gathers(q)

        if prefetch_pred is None:
            launch()
        else:
            pl.when(prefetch_pred)(launch)

    pltpu.sync_copy(src_hbm.at[pl.ds(row_base, INNER)], src_v.at[0])
    pltpu.sync_copy(dst_hbm.at[pl.ds(row_base, INNER)], dst_v.at[0])
    pltpu.sync_copy(val_hbm.at[pl.ds(row_base, INNER)], val_v.at[0])
    xform_and_fire_gathers(0)

    def super_body2(g2, carry):
        stage(2 * g2, 0, None)
        stage(2 * g2 + 1, 1, g2 < n_super // 2 - 1)
        return carry

    lax.fori_loop(0, n_super // 2, super_body2, 0)
    plsc.subcore_barrier()

    pltpu.sync_copy(agg_sh.at[pl.ds(r0, ROWS_PER_SUB)],
                    out_hbm.at[pl.ds(c * N + r0, ROWS_PER_SUB)])


@functools.partial(jax.jit, static_argnames=("n_super",))
def _sc_aggregate(emb_r, src2, dst2, val2, n_super):
    mesh = plsc.VectorSubcoreMesh(core_axis_name="c", subcore_axis_name="s",
                                  num_cores=NC, num_subcores=NS)
    return pl.kernel(
        functools.partial(_agg_body, n_super=n_super),
        out_type=jax.ShapeDtypeStruct((2 * N, HALF), jnp.float32),
        mesh=mesh,
        scratch_types=[
            pltpu.VMEM_SHARED((N, HALF), jnp.float32),
            pltpu.VMEM((2, INNER, BLK), jnp.int32),
            pltpu.VMEM((2, INNER, BLK), jnp.int32),
            pltpu.VMEM((2, INNER, BLK), jnp.float32),
            pltpu.VMEM((INNER, BLK, HALF), jnp.float32),
            pltpu.VMEM((ZROWS, HALF), jnp.float32),
            pltpu.SemaphoreType.DMA,
            pltpu.SemaphoreType.DMA,
            pltpu.SemaphoreType.DMA,
        ],
        compiler_params=pltpu.CompilerParams(use_tc_tiling_on_sc=False),
    )(emb_r, src2, dst2, val2)


def _linear_kernel(x_ref, w_ref, b_ref, o_ref):
    h = jnp.dot(x_ref[...], w_ref[...], preferred_element_type=jnp.float32)
    o_ref[...] = jnp.maximum(h + b_ref[...], 0.0)


def _linear(agg_r, W, b):
    RB = 1000
    nrows = 2 * N // 8
    w_bd = jnp.kron(jnp.eye(4, dtype=jnp.float32), W.T)
    b_t = jnp.tile(b, 4).reshape(1, 128)
    x = agg_r.reshape(nrows, 128)
    y = pl.pallas_call(
        _linear_kernel,
        grid=(nrows // RB,),
        in_specs=[
            pl.BlockSpec((RB, 128), lambda i: (i, 0)),
            pl.BlockSpec((128, 128), lambda i: (0, 0)),
            pl.BlockSpec((1, 128), lambda i: (0, 0)),
        ],
        out_specs=pl.BlockSpec((RB, 128), lambda i: (i, 0)),
        out_shape=jax.ShapeDtypeStruct((nrows, 128), jnp.float32),
    )(x, w_bd, b_t)
    return y


def kernel(adj_indices, adj_values, emb_table, W0, b0, W1, b1):
    E = adj_values.shape[0]
    e_unit = NS * SUPER
    E_pad = -(-E // e_unit) * e_unit
    pad = E_pad - E
    if pad:
        pad_idx = (jnp.arange(pad, dtype=jnp.int32) * 97) % N
        src = jnp.concatenate([adj_indices[1], pad_idx])
        dst = jnp.concatenate([adj_indices[0], pad_idx])
        val = jnp.concatenate([adj_values, jnp.zeros((pad,), jnp.float32)])
    else:
        src, dst, val = adj_indices[1], adj_indices[0], adj_values
    src2 = src.reshape(-1, BLK)
    dst2 = dst.reshape(-1, BLK)
    val2 = val.reshape(-1, BLK)
    n_super = E_pad // (NS * SUPER)

    emb_r = emb_table.reshape(2 * N, HALF)
    agg1 = _sc_aggregate(emb_r, src2, dst2, val2, n_super)
    emb1 = _linear(agg1, W0, b0)
    agg2 = _sc_aggregate(emb1.reshape(2 * N, HALF), src2, dst2, val2, n_super)
    return _linear(agg2, W1, b1).reshape(N, D)
